# Initial kernel scaffold; baseline (speedup 1.0000x reference)
import jax
import jax.numpy as jnp
from jax.experimental import pallas as pl


def kernel(x, pos, seq, ori, batch, edge_index, features, chem, geo, surf2struct, params):
    raise NotImplementedError("write your pallas kernel here")


# speedup vs baseline: 1.2071x; 1.2071x over previous
import functools

import jax
import jax.numpy as jnp
from jax import lax
from jax.experimental import pallas as pl
from jax.experimental.pallas import tpu as pltpu
from jax.experimental.pallas import tpu_sc as plsc

N, E, M, B = 10000, 320000, 20000, 8
CH, EMB, W, KC = 512, 16, 128, 24


def _lrelu(x, s):
    return jnp.where(x >= 0, x, s * x)


def _dot(a, b):
    return jnp.dot(a, b, preferred_element_type=jnp.float32)


def _surf_body(chem_ref, geo_ref, feat_ref, wmp1, bmp1, g1, b1, wmp2, bmp2,
               wcat, bcat, wkp1, bkp1, wkp2, bkp2, out_ref):
    bm = chem_ref.shape[0]
    c = chem_ref[...].reshape(bm * 16, 15)
    h = _dot(c, wmp1[...]) + bmp1[...]
    h = h * g1[...] + b1[...]
    h = _lrelu(h, 0.1)
    h = _dot(h, wmp2[...]) + bmp2[...]
    cf = jnp.max(h.reshape(bm, 16, 21), axis=1)
    wc = wcat[...]
    es = (_dot(geo_ref[...], wc[0:10]) + _dot(cf, wc[10:31])
          + _dot(feat_ref[...], wc[31:32]) + bcat[...])
    t = _lrelu(_dot(es, wkp1[...]) + bkp1[...], 0.1)
    out_ref[...] = _dot(t, wkp2[...]) + bkp2[...]


def _full_spec(a):
    r = len(a.shape)
    return pl.BlockSpec(a.shape, lambda *_, _r=r: (0,) * _r)


def _surface(chem, geo, features, p):
    bm = 800
    grid = (M // bm,)
    ws = [p['W_mp1'], p['b_mp1'], p['mp_bng'], p['mp_bnb'], p['W_mp2'], p['b_mp2'],
          p['W_cat'], p['b_cat'], p['W_kp1'], p['b_kp1'], p['W_kp2'], p['b_kp2']]
    return pl.pallas_call(
        _surf_body,
        grid=grid,
        in_specs=[pl.BlockSpec((bm, 16, 15), lambda i: (i, 0, 0)),
                  pl.BlockSpec((bm, 10), lambda i: (i, 0)),
                  pl.BlockSpec((bm, 1), lambda i: (i, 0))] + [_full_spec(w) for w in ws],
        out_specs=pl.BlockSpec((bm, 32), lambda i: (i, 0)),
        out_shape=jax.ShapeDtypeStruct((M, 32), jnp.float32),
    )(chem, geo, features, *ws)


def _kern_body(kin_ref, wk1, bk1, wk2, out_ref):
    a = _dot(kin_ref[...], wk1[...]) + bk1[...]
    a = _lrelu(a, 0.1)
    out_ref[...] = _dot(a, wk2[...])


def _edge_kern(kin, wk1, bk1, wk2):
    ke = 3200
    return pl.pallas_call(
        _kern_body,
        grid=(E // ke,),
        in_specs=[pl.BlockSpec((ke, 4), lambda i: (i, 0)),
                  _full_spec(wk1), _full_spec(bk1), _full_spec(wk2)],
        out_specs=pl.BlockSpec((ke, W), lambda i: (i, 0)),
        out_shape=jax.ShapeDtypeStruct((E, W), jnp.float32),
    )(kin, wk1, bk1, wk2)


def _pre0_body(x_ref, fuse_ref, surf_g_ref, emb, wfus, bfus, g1, b1, inw, inb,
               g2, b2, gid, bid, idw, z_ref, id_ref):
    xi = x_ref[...]
    onehot = (xi == lax.broadcasted_iota(jnp.int32, (1, 21), 1)).astype(jnp.float32)
    h = _dot(onehot, emb[...])
    fuse = _dot(surf_g_ref[...], wfus[...]) + bfus[...]
    del fuse_ref
    z = _lrelu(h * g1[...] + b1[...], 0.2)
    z = _dot(z, inw[...]) + inb[...]
    z = _lrelu(z * g2[...] + b2[...], 0.2)
    z_ref[...] = z + fuse
    id_ref[...] = _dot(_lrelu(h * gid[...] + bid[...], 0.1), idw[...])


def _pre0(x, surf_g, p):
    bn = 1000
    pre = 'b0'
    ws = [p['emb'], p['W_fus'], p['b_fus'],
          p[pre + '_in_bng1'], p[pre + '_in_bnb1'], p[pre + '_in_W'], p[pre + '_in_b'],
          p[pre + '_in_bng2'], p[pre + '_in_bnb2'],
          p[pre + '_id_bng'], p[pre + '_id_bnb'], p[pre + '_id_W']]
    dummy = jnp.zeros((N, 1), jnp.float32)
    return pl.pallas_call(
        _pre0_body,
        grid=(N // bn,),
        in_specs=[pl.BlockSpec((bn, 1), lambda i: (i, 0)),
                  pl.BlockSpec((bn, 1), lambda i: (i, 0)),
                  pl.BlockSpec((bn, 32), lambda i: (i, 0))] + [_full_spec(w) for w in ws],
        out_specs=[pl.BlockSpec((bn, W), lambda i: (i, 0)),
                   pl.BlockSpec((bn, CH), lambda i: (i, 0))],
        out_shape=[jax.ShapeDtypeStruct((N, W), jnp.float32),
                   jax.ShapeDtypeStruct((N, CH), jnp.float32)],
    )(x.reshape(N, 1), dummy, surf_g, *ws)


def _pre_body(h_ref, g1, b1, inw, inb, g2, b2, z_ref):
    z = _lrelu(h_ref[...] * g1[...] + b1[...], 0.2)
    z = _dot(z, inw[...]) + inb[...]
    z_ref[...] = _lrelu(z * g2[...] + b2[...], 0.2)


def _pre(h, p, i):
    n = h.shape[0]
    bn = min(n, 1024)
    pre = 'b%d' % i
    ws = [p[pre + '_in_bng1'], p[pre + '_in_bnb1'], p[pre + '_in_W'], p[pre + '_in_b'],
          p[pre + '_in_bng2'], p[pre + '_in_bnb2']]
    return pl.pallas_call(
        _pre_body,
        grid=(pl.cdiv(n, bn),),
        in_specs=[pl.BlockSpec((bn, CH), lambda i: (i, 0))] + [_full_spec(w) for w in ws],
        out_specs=pl.BlockSpec((bn, W), lambda i: (i, 0)),
        out_shape=jax.ShapeDtypeStruct((n, W), jnp.float32),
    )(h, *ws)


def _post_body(p_ref, deg_ref, id_ref, wc, go, bo, outw, h_ref, *, pool):
    agg = p_ref[...] / jnp.maximum(deg_ref[...], 1.0)
    z = _dot(agg, wc[...])
    z = _lrelu(z * go[...] + bo[...], 0.1)
    hn = _dot(z, outw[...]) + id_ref[...]
    if pool:
        bn = hn.shape[0]
        hp = hn.reshape(bn // 2, 2, CH)
        h_ref[...] = (hp[:, 0, :] + hp[:, 1, :]) * 0.5
    else:
        h_ref[...] = hn


def _post(psum, deg, ident, p, i, pool):
    n = psum.shape[0]
    bn = min(n, 1024)
    pre = 'b%d' % i
    ws = [p[pre + '_Wc'], p[pre + '_out_bng'], p[pre + '_out_bnb'], p[pre + '_out_W']]
    nout = n // 2 if pool else n
    bout = bn // 2 if pool else bn
    return pl.pallas_call(
        functools.partial(_post_body, pool=pool),
        grid=(pl.cdiv(n, bn),),
        in_specs=[pl.BlockSpec((bn, W), lambda i: (i, 0)),
                  pl.BlockSpec((bn, 1), lambda i: (i, 0)),
                  pl.BlockSpec((bn, CH), lambda i: (i, 0))] + [_full_spec(w) for w in ws],
        out_specs=pl.BlockSpec((bout, CH), lambda i: (i, 0)),
        out_shape=jax.ShapeDtypeStruct((nout, CH), jnp.float32),
    )(psum, deg, ident, *ws)


def _head_body(h_ref, batch_ref, g1, b1, w1, bb1, g2, b2, w2, bb2, out_ref):
    bid = batch_ref[...]
    mask = (bid == lax.broadcasted_iota(jnp.int32, (1, B), 1)).astype(jnp.float32)
    s = lax.dot_general(mask, h_ref[...], (((0,), (0,)), ((), ())),
                        preferred_element_type=jnp.float32)
    cnt = jnp.sum(mask, axis=0)[:, None]
    h = s / jnp.maximum(cnt, 1.0)
    z = _lrelu(h * g1[...] + b1[...], 0.2)
    z = _dot(z, w1[...]) + bb1[...]
    z = _lrelu(z * g2[...] + b2[...], 0.2)
    out_ref[...] = _dot(z, w2[...]) + bb2[...]


def _head(h, batch, p):
    n = h.shape[0]
    ws = [p['cls_bng1'], p['cls_bnb1'], p['cls_W1'], p['cls_b1'],
          p['cls_bng2'], p['cls_bnb2'], p['cls_W2'], p['cls_b2']]
    return pl.pallas_call(
        _head_body,
        in_specs=[pl.BlockSpec((n, CH), lambda: (0, 0)),
                  pl.BlockSpec((n, 1), lambda: (0, 0))] + [_full_spec(w) for w in ws],
        out_specs=pl.BlockSpec((B, 384), lambda: (0, 0)),
        out_shape=jax.ShapeDtypeStruct((B, 384), jnp.float32),
    )(h, batch.reshape(n, 1), *ws)


def kernel(x, pos, seq, ori, batch, edge_index, features, chem, geo,
           surf2struct, params):
    p = params
    out_surf = _surface(chem, geo, features, p)
    surf_g = jnp.take(out_surf, surf2struct, axis=0)

    g = jnp.concatenate([pos, seq, ori], axis=-1)
    src0, dst0 = edge_index[0], edge_index[1]

    h = None
    ident = None
    n = N
    for i in range(8):
        lvl = (i // 2)
        if i % 2 == 0:
            e_src = src0 // (1 << lvl)
            e_dst = dst0 // (1 << lvl)
            gs = jnp.take(g, e_src, axis=0)
            gd = jnp.take(g, e_dst, axis=0)
            rel = gd[:, 0:3] - gs[:, 0:3]
            rl0 = gd[:, 4] * rel[:, 0] + gd[:, 5] * rel[:, 1] + gd[:, 6] * rel[:, 2]
            rl1 = gd[:, 7] * rel[:, 0] + gd[:, 8] * rel[:, 1] + gd[:, 9] * rel[:, 2]
            rl2 = gd[:, 10] * rel[:, 0] + gd[:, 11] * rel[:, 1] + gd[:, 12] * rel[:, 2]
            kin = jnp.stack([rl0, rl1, rl2, gd[:, 3] - gs[:, 3]], axis=-1)
            deg = jax.ops.segment_sum(jnp.ones((E, 1), jnp.float32), e_dst,
                                      num_segments=n)

        pre = 'b%d' % i
        if i == 0:
            z, ident = _pre0(x, surf_g, p)
        else:
            z = _pre(h, p, i)
            ident = h

        kern = _edge_kern(kin, p[pre + '_Wk1'], p[pre + '_bk1'], p[pre + '_Wk2'])
        msg = jnp.take(z, e_src, axis=0) * kern
        psum = jax.ops.segment_sum(msg, e_dst, num_segments=n)

        pool = (i % 2 == 1) and i != 7
        h = _post(psum, deg, ident, p, i, pool)
        if pool:
            g = (g[0::2] + g[1::2]) * 0.5
            batch = batch[0::2]
            n = n // 2

    return _head(h, batch, p)


# speedup vs baseline: 1.7460x; 1.4465x over previous
import functools

import jax
import jax.numpy as jnp
from jax import lax
from jax.experimental import pallas as pl
from jax.experimental.pallas import tpu as pltpu
from jax.experimental.pallas import tpu_sc as plsc

N, E, M, B = 10000, 320000, 20000, 8
CH, EMB, W, KC = 512, 16, 128, 24


def _lrelu(x, s):
    return jnp.where(x >= 0, x, s * x)


_KE = 80
_NT = 32
_ZB = 8


def _npad(n):
    return ((n + 127) // 128) * 128


def _conv_sc_make(n):
    n_pad = _npad(n)
    rpt = n_pad // 16
    per_tile = E // _NT
    chunks = per_tile // _KE
    mesh = plsc.VectorSubcoreMesh(core_axis_name="c", subcore_axis_name="s")

    def body(z_hbm, kern_hbm, src_hbm, dst_hbm, out_hbm,
             idx_s, idx_d, rows, kv, zbuf, accum, sem):
        cid = lax.axis_index("c")
        sid = lax.axis_index("s")
        wid = sid * 2 + cid
        zero16 = jnp.zeros((16,), jnp.float32)

        def zrow(r, _):
            for c in range(8):
                zbuf[r, pl.ds(c * 16, 16)] = zero16
            return 0
        lax.fori_loop(0, _ZB, zrow, 0)
        row0 = sid * rpt
        for b in range(rpt // _ZB):
            pltpu.sync_copy(zbuf, accum.at[pl.ds(row0 + b * _ZB, _ZB)])
        plsc.subcore_barrier()

        def chunk(k, _):
            off = wid * per_tile + k * _KE
            pltpu.sync_copy(src_hbm.at[pl.ds(off, _KE)], idx_s)
            pltpu.sync_copy(dst_hbm.at[pl.ds(off, _KE)], idx_d)
            pltpu.async_copy(z_hbm.at[idx_s], rows, sem).wait()
            pltpu.sync_copy(kern_hbm.at[pl.ds(off, _KE)], kv)

            def mul(r, _):
                for c in range(8):
                    s = pl.ds(c * 16, 16)
                    rows[r, s] = rows[r, s] * kv[r, s]
                return 0
            lax.fori_loop(0, _KE, mul, 0)
            pltpu.sync_copy(rows, accum.at[idx_d], add=True)
            return 0
        lax.fori_loop(0, chunks, chunk, 0)
        plsc.subcore_barrier()
        pltpu.sync_copy(accum.at[pl.ds(row0, rpt)],
                        out_hbm.at[pl.ds(cid * n_pad + row0, rpt)])

    return pl.kernel(
        body, mesh=mesh,
        out_type=[jax.ShapeDtypeStruct((2 * n_pad, 128), jnp.float32)],
        scratch_types=[
            pltpu.VMEM((_KE,), jnp.int32),
            pltpu.VMEM((_KE,), jnp.int32),
            pltpu.VMEM((_KE, 128), jnp.float32),
            pltpu.VMEM((_KE, 128), jnp.float32),
            pltpu.VMEM((_ZB, 128), jnp.float32),
            pltpu.VMEM_SHARED((n_pad, 128), jnp.float32),
            pltpu.SemaphoreType.DMA,
        ])


def _conv_sc(z, kern, src, dst, n):
    (out,) = _conv_sc_make(n)(z, kern, src, dst)
    n_pad = _npad(n)
    return out[:n], out[n_pad:n_pad + n]


def _deg_sc_make(n):
    n_pad = _npad(n)
    rpt = n_pad // 16
    per_tile = E // _NT
    chunks = per_tile // _KE
    mesh = plsc.VectorSubcoreMesh(core_axis_name="c", subcore_axis_name="s")

    def body(dst_hbm, out_hbm, idx_d, ones_v, zbuf, accum):
        cid = lax.axis_index("c")
        sid = lax.axis_index("s")
        wid = sid * 2 + cid
        zero16 = jnp.zeros((16,), jnp.float32)
        one16 = jnp.ones((16,), jnp.float32)

        def orow(r, _):
            for c in range(8):
                ones_v[r, pl.ds(c * 16, 16)] = one16
            return 0
        lax.fori_loop(0, _KE, orow, 0)

        def zrow(r, _):
            for c in range(8):
                zbuf[r, pl.ds(c * 16, 16)] = zero16
            return 0
        lax.fori_loop(0, _ZB, zrow, 0)
        row0 = sid * rpt
        for b in range(rpt // _ZB):
            pltpu.sync_copy(zbuf, accum.at[pl.ds(row0 + b * _ZB, _ZB)])
        plsc.subcore_barrier()

        def chunk(k, _):
            off = wid * per_tile + k * _KE
            pltpu.sync_copy(dst_hbm.at[pl.ds(off, _KE)], idx_d)
            pltpu.sync_copy(ones_v, accum.at[idx_d], add=True)
            return 0
        lax.fori_loop(0, chunks, chunk, 0)
        plsc.subcore_barrier()
        pltpu.sync_copy(accum.at[pl.ds(row0, rpt)],
                        out_hbm.at[pl.ds(cid * n_pad + row0, rpt)])

    return pl.kernel(
        body, mesh=mesh,
        out_type=[jax.ShapeDtypeStruct((2 * n_pad, 128), jnp.float32)],
        scratch_types=[
            pltpu.VMEM((_KE,), jnp.int32),
            pltpu.VMEM((_KE, 128), jnp.float32),
            pltpu.VMEM((_ZB, 128), jnp.float32),
            pltpu.VMEM_SHARED((n_pad, 128), jnp.float32),
        ])


def _deg_sc(dst, n):
    (out,) = _deg_sc_make(n)(dst)
    n_pad = _npad(n)
    return out[:n] + out[n_pad:n_pad + n]


def _gather_sc_make(v, k):
    per_tile = k // _NT
    chunks = per_tile // _KE
    mesh = plsc.VectorSubcoreMesh(core_axis_name="c", subcore_axis_name="s")

    def body(table_hbm, idx_hbm, out_hbm, idx_v, rows, sem):
        cid = lax.axis_index("c")
        sid = lax.axis_index("s")
        wid = sid * 2 + cid

        def chunk(j, _):
            off = wid * per_tile + j * _KE
            pltpu.sync_copy(idx_hbm.at[pl.ds(off, _KE)], idx_v)
            pltpu.async_copy(table_hbm.at[idx_v], rows, sem).wait()
            pltpu.sync_copy(rows, out_hbm.at[pl.ds(off, _KE)])
            return 0
        lax.fori_loop(0, chunks, chunk, 0)

    return pl.kernel(
        body, mesh=mesh,
        out_type=[jax.ShapeDtypeStruct((k, 128), jnp.float32)],
        scratch_types=[
            pltpu.VMEM((_KE,), jnp.int32),
            pltpu.VMEM((_KE, 128), jnp.float32),
            pltpu.SemaphoreType.DMA,
        ])


def _gather_sc(table, idx):
    k0 = idx.shape[0]
    kq = _NT * _KE
    k_pad = ((k0 + kq - 1) // kq) * kq
    if k_pad != k0:
        idx = jnp.pad(idx, (0, k_pad - k0))
    (out,) = _gather_sc_make(table.shape[0], k_pad)(table, idx)
    return out[:k0]


def _dot(a, b):
    return jnp.dot(a, b, preferred_element_type=jnp.float32)


def _surf_body(chem_ref, geo_ref, feat_ref, wmp1, bmp1, g1, b1, wmp2, bmp2,
               wcat, bcat, wkp1, bkp1, wkp2, bkp2, out_ref):
    bm = chem_ref.shape[0]
    c = chem_ref[...].reshape(bm * 16, 15)
    h = _dot(c, wmp1[...]) + bmp1[...]
    h = h * g1[...] + b1[...]
    h = _lrelu(h, 0.1)
    h = _dot(h, wmp2[...]) + bmp2[...]
    cf = jnp.max(h.reshape(bm, 16, 21), axis=1)
    wc = wcat[...]
    es = (_dot(geo_ref[...], wc[0:10]) + _dot(cf, wc[10:31])
          + _dot(feat_ref[...], wc[31:32]) + bcat[...])
    t = _lrelu(_dot(es, wkp1[...]) + bkp1[...], 0.1)
    r = _dot(t, wkp2[...]) + bkp2[...]
    out_ref[...] = jnp.concatenate(
        [r, jnp.zeros((bm, 96), jnp.float32)], axis=1)


def _full_spec(a):
    r = len(a.shape)
    return pl.BlockSpec(a.shape, lambda *_, _r=r: (0,) * _r)


def _surface(chem, geo, features, p):
    bm = 800
    grid = (M // bm,)
    ws = [p['W_mp1'], p['b_mp1'], p['mp_bng'], p['mp_bnb'], p['W_mp2'], p['b_mp2'],
          p['W_cat'], p['b_cat'], p['W_kp1'], p['b_kp1'], p['W_kp2'], p['b_kp2']]
    return pl.pallas_call(
        _surf_body,
        grid=grid,
        in_specs=[pl.BlockSpec((bm, 16, 15), lambda i: (i, 0, 0)),
                  pl.BlockSpec((bm, 10), lambda i: (i, 0)),
                  pl.BlockSpec((bm, 1), lambda i: (i, 0))] + [_full_spec(w) for w in ws],
        out_specs=pl.BlockSpec((bm, 128), lambda i: (i, 0)),
        out_shape=jax.ShapeDtypeStruct((M, 128), jnp.float32),
    )(chem, geo, features, *ws)


def _kern_body(kin_ref, wk1, bk1, wk2, out_ref):
    a = _dot(kin_ref[...], wk1[...]) + bk1[...]
    a = _lrelu(a, 0.1)
    out_ref[...] = _dot(a, wk2[...])


def _edge_kern(kin, wk1, bk1, wk2):
    ke = 3200
    return pl.pallas_call(
        _kern_body,
        grid=(E // ke,),
        in_specs=[pl.BlockSpec((ke, 4), lambda i: (i, 0)),
                  _full_spec(wk1), _full_spec(bk1), _full_spec(wk2)],
        out_specs=pl.BlockSpec((ke, W), lambda i: (i, 0)),
        out_shape=jax.ShapeDtypeStruct((E, W), jnp.float32),
    )(kin, wk1, bk1, wk2)


def _pre0_body(x_ref, fuse_ref, surf_g_ref, emb, wfus, bfus, g1, b1, inw, inb,
               g2, b2, gid, bid, idw, z_ref, id_ref):
    xi = x_ref[...]
    onehot = (xi == lax.broadcasted_iota(jnp.int32, (1, 21), 1)).astype(jnp.float32)
    h = _dot(onehot, emb[...])
    fuse = _dot(surf_g_ref[...][:, 0:32], wfus[...]) + bfus[...]
    del fuse_ref
    z = _lrelu(h * g1[...] + b1[...], 0.2)
    z = _dot(z, inw[...]) + inb[...]
    z = _lrelu(z * g2[...] + b2[...], 0.2)
    z_ref[...] = z + fuse
    id_ref[...] = _dot(_lrelu(h * gid[...] + bid[...], 0.1), idw[...])


def _pre0(x, surf_g, p):
    bn = 1000
    pre = 'b0'
    ws = [p['emb'], p['W_fus'], p['b_fus'],
          p[pre + '_in_bng1'], p[pre + '_in_bnb1'], p[pre + '_in_W'], p[pre + '_in_b'],
          p[pre + '_in_bng2'], p[pre + '_in_bnb2'],
          p[pre + '_id_bng'], p[pre + '_id_bnb'], p[pre + '_id_W']]
    dummy = jnp.zeros((N, 1), jnp.float32)
    return pl.pallas_call(
        _pre0_body,
        grid=(N // bn,),
        in_specs=[pl.BlockSpec((bn, 1), lambda i: (i, 0)),
                  pl.BlockSpec((bn, 1), lambda i: (i, 0)),
                  pl.BlockSpec((bn, 128), lambda i: (i, 0))] + [_full_spec(w) for w in ws],
        out_specs=[pl.BlockSpec((bn, W), lambda i: (i, 0)),
                   pl.BlockSpec((bn, CH), lambda i: (i, 0))],
        out_shape=[jax.ShapeDtypeStruct((N, W), jnp.float32),
                   jax.ShapeDtypeStruct((N, CH), jnp.float32)],
    )(x.reshape(N, 1), dummy, surf_g, *ws)


def _pre_body(h_ref, g1, b1, inw, inb, g2, b2, z_ref):
    z = _lrelu(h_ref[...] * g1[...] + b1[...], 0.2)
    z = _dot(z, inw[...]) + inb[...]
    z_ref[...] = _lrelu(z * g2[...] + b2[...], 0.2)


def _pre(h, p, i):
    n = h.shape[0]
    bn = min(n, 1024)
    pre = 'b%d' % i
    ws = [p[pre + '_in_bng1'], p[pre + '_in_bnb1'], p[pre + '_in_W'], p[pre + '_in_b'],
          p[pre + '_in_bng2'], p[pre + '_in_bnb2']]
    return pl.pallas_call(
        _pre_body,
        grid=(pl.cdiv(n, bn),),
        in_specs=[pl.BlockSpec((bn, CH), lambda i: (i, 0))] + [_full_spec(w) for w in ws],
        out_specs=pl.BlockSpec((bn, W), lambda i: (i, 0)),
        out_shape=jax.ShapeDtypeStruct((n, W), jnp.float32),
    )(h, *ws)


def _post_body(p0_ref, p1_ref, deg_ref, id_ref, wc, go, bo, outw, h_ref, *, pool):
    agg = (p0_ref[...] + p1_ref[...]) / jnp.maximum(deg_ref[...], 1.0)
    z = _dot(agg, wc[...])
    z = _lrelu(z * go[...] + bo[...], 0.1)
    hn = _dot(z, outw[...]) + id_ref[...]
    if pool:
        bn = hn.shape[0]
        hp = hn.reshape(bn // 2, 2, CH)
        h_ref[...] = (hp[:, 0, :] + hp[:, 1, :]) * 0.5
    else:
        h_ref[...] = hn


def _post(p0, p1, deg, ident, p, i, pool):
    n = p0.shape[0]
    bn = min(n, 1024)
    pre = 'b%d' % i
    ws = [p[pre + '_Wc'], p[pre + '_out_bng'], p[pre + '_out_bnb'], p[pre + '_out_W']]
    nout = n // 2 if pool else n
    bout = bn // 2 if pool else bn
    return pl.pallas_call(
        functools.partial(_post_body, pool=pool),
        grid=(pl.cdiv(n, bn),),
        in_specs=[pl.BlockSpec((bn, W), lambda i: (i, 0)),
                  pl.BlockSpec((bn, W), lambda i: (i, 0)),
                  pl.BlockSpec((bn, W), lambda i: (i, 0)),
                  pl.BlockSpec((bn, CH), lambda i: (i, 0))] + [_full_spec(w) for w in ws],
        out_specs=pl.BlockSpec((bout, CH), lambda i: (i, 0)),
        out_shape=jax.ShapeDtypeStruct((nout, CH), jnp.float32),
    )(p0, p1, deg, ident, *ws)


def _head_body(h_ref, batch_ref, g1, b1, w1, bb1, g2, b2, w2, bb2, out_ref):
    bid = batch_ref[...]
    mask = (bid == lax.broadcasted_iota(jnp.int32, (1, B), 1)).astype(jnp.float32)
    s = lax.dot_general(mask, h_ref[...], (((0,), (0,)), ((), ())),
                        preferred_element_type=jnp.float32)
    cnt = jnp.sum(mask, axis=0)[:, None]
    h = s / jnp.maximum(cnt, 1.0)
    z = _lrelu(h * g1[...] + b1[...], 0.2)
    z = _dot(z, w1[...]) + bb1[...]
    z = _lrelu(z * g2[...] + b2[...], 0.2)
    out_ref[...] = _dot(z, w2[...]) + bb2[...]


def _head(h, batch, p):
    n = h.shape[0]
    ws = [p['cls_bng1'], p['cls_bnb1'], p['cls_W1'], p['cls_b1'],
          p['cls_bng2'], p['cls_bnb2'], p['cls_W2'], p['cls_b2']]
    return pl.pallas_call(
        _head_body,
        in_specs=[pl.BlockSpec((n, CH), lambda: (0, 0)),
                  pl.BlockSpec((n, 1), lambda: (0, 0))] + [_full_spec(w) for w in ws],
        out_specs=pl.BlockSpec((B, 384), lambda: (0, 0)),
        out_shape=jax.ShapeDtypeStruct((B, 384), jnp.float32),
    )(h, batch.reshape(n, 1), *ws)


def kernel(x, pos, seq, ori, batch, edge_index, features, chem, geo,
           surf2struct, params):
    p = params
    out_surf = _surface(chem, geo, features, p)
    surf_g = _gather_sc(out_surf, surf2struct)

    g = jnp.concatenate([pos, seq, ori], axis=-1)
    src0, dst0 = edge_index[0], edge_index[1]

    h = None
    ident = None
    n = N
    for i in range(8):
        lvl = (i // 2)
        if i % 2 == 0:
            e_src = src0 // (1 << lvl)
            e_dst = dst0 // (1 << lvl)
            gs = jnp.take(g, e_src, axis=0)
            gd = jnp.take(g, e_dst, axis=0)
            rel = gd[:, 0:3] - gs[:, 0:3]
            rl0 = gd[:, 4] * rel[:, 0] + gd[:, 5] * rel[:, 1] + gd[:, 6] * rel[:, 2]
            rl1 = gd[:, 7] * rel[:, 0] + gd[:, 8] * rel[:, 1] + gd[:, 9] * rel[:, 2]
            rl2 = gd[:, 10] * rel[:, 0] + gd[:, 11] * rel[:, 1] + gd[:, 12] * rel[:, 2]
            kin = jnp.stack([rl0, rl1, rl2, gd[:, 3] - gs[:, 3]], axis=-1)
            if i == 0:
                deg = _deg_sc(e_dst, n)

        pre = 'b%d' % i
        if i == 0:
            z, ident = _pre0(x, surf_g, p)
        else:
            z = _pre(h, p, i)
            ident = h

        kern = _edge_kern(kin, p[pre + '_Wk1'], p[pre + '_bk1'], p[pre + '_Wk2'])
        p0, p1 = _conv_sc(z, kern, e_src, e_dst, n)

        pool = (i % 2 == 1) and i != 7
        h = _post(p0, p1, deg, ident, p, i, pool)
        if pool:
            g = (g[0::2] + g[1::2]) * 0.5
            deg = deg[0::2] + deg[1::2]
            batch = batch[0::2]
            n = n // 2

    return _head(h, batch, p)


# speedup vs baseline: 4.7638x; 2.7284x over previous
import functools

import jax
import jax.numpy as jnp
from jax import lax
from jax.experimental import pallas as pl
from jax.experimental.pallas import tpu as pltpu
from jax.experimental.pallas import tpu_sc as plsc

N, E, M, B = 10000, 320000, 20000, 8
CH, EMB, W, KC = 512, 16, 128, 24


def _lrelu(x, s):
    return jnp.where(x >= 0, x, s * x)


_KE = 80
_NT = 32
_ZB = 8


def _npad(n):
    return ((n + 127) // 128) * 128


def _conv_sc_make(n):
    n_pad = _npad(n)
    rpt = n_pad // 16
    per_tile = E // _NT
    chunks = per_tile // _KE
    mesh = plsc.VectorSubcoreMesh(core_axis_name="c", subcore_axis_name="s")

    def body(z_hbm, kern_hbm, src_hbm, dst_hbm, out_hbm,
             idx_s, idx_d, rows, kv, zbuf, accum, sem):
        cid = lax.axis_index("c")
        sid = lax.axis_index("s")
        wid = sid * 2 + cid
        zero16 = jnp.zeros((16,), jnp.float32)

        def zrow(r, _):
            for c in range(8):
                zbuf[r, pl.ds(c * 16, 16)] = zero16
            return 0
        lax.fori_loop(0, _ZB, zrow, 0)
        row0 = sid * rpt
        for b in range(rpt // _ZB):
            pltpu.sync_copy(zbuf, accum.at[pl.ds(row0 + b * _ZB, _ZB)])
        plsc.subcore_barrier()

        def chunk(k, _):
            off = wid * per_tile + k * _KE
            pltpu.sync_copy(src_hbm.at[pl.ds(off, _KE)], idx_s)
            pltpu.sync_copy(dst_hbm.at[pl.ds(off, _KE)], idx_d)
            pltpu.async_copy(z_hbm.at[idx_s], rows, sem).wait()
            pltpu.sync_copy(kern_hbm.at[pl.ds(off, _KE)], kv)

            def mul(r, _):
                for c in range(8):
                    s = pl.ds(c * 16, 16)
                    rows[r, s] = rows[r, s] * kv[r, s]
                return 0
            lax.fori_loop(0, _KE, mul, 0)
            pltpu.sync_copy(rows, accum.at[idx_d], add=True)
            return 0
        lax.fori_loop(0, chunks, chunk, 0)
        plsc.subcore_barrier()
        pltpu.sync_copy(accum.at[pl.ds(row0, rpt)],
                        out_hbm.at[pl.ds(cid * n_pad + row0, rpt)])

    return pl.kernel(
        body, mesh=mesh,
        out_type=[jax.ShapeDtypeStruct((2 * n_pad, 128), jnp.float32)],
        scratch_types=[
            pltpu.VMEM((_KE,), jnp.int32),
            pltpu.VMEM((_KE,), jnp.int32),
            pltpu.VMEM((_KE, 128), jnp.float32),
            pltpu.VMEM((_KE, 128), jnp.float32),
            pltpu.VMEM((_ZB, 128), jnp.float32),
            pltpu.VMEM_SHARED((n_pad, 128), jnp.float32),
            pltpu.SemaphoreType.DMA,
        ])


def _conv_sc(z, kern, src, dst, n):
    (out,) = _conv_sc_make(n)(z, kern, src, dst)
    n_pad = _npad(n)
    return out[:n], out[n_pad:n_pad + n]


def _deg_sc_make(n):
    n_pad = _npad(n)
    rpt = n_pad // 16
    per_tile = E // _NT
    chunks = per_tile // _KE
    mesh = plsc.VectorSubcoreMesh(core_axis_name="c", subcore_axis_name="s")

    def body(dst_hbm, out_hbm, idx_d, ones_v, zbuf, accum):
        cid = lax.axis_index("c")
        sid = lax.axis_index("s")
        wid = sid * 2 + cid
        zero16 = jnp.zeros((16,), jnp.float32)
        one16 = jnp.ones((16,), jnp.float32)

        def orow(r, _):
            for c in range(8):
                ones_v[r, pl.ds(c * 16, 16)] = one16
            return 0
        lax.fori_loop(0, _KE, orow, 0)

        def zrow(r, _):
            for c in range(8):
                zbuf[r, pl.ds(c * 16, 16)] = zero16
            return 0
        lax.fori_loop(0, _ZB, zrow, 0)
        row0 = sid * rpt
        for b in range(rpt // _ZB):
            pltpu.sync_copy(zbuf, accum.at[pl.ds(row0 + b * _ZB, _ZB)])
        plsc.subcore_barrier()

        def chunk(k, _):
            off = wid * per_tile + k * _KE
            pltpu.sync_copy(dst_hbm.at[pl.ds(off, _KE)], idx_d)
            pltpu.sync_copy(ones_v, accum.at[idx_d], add=True)
            return 0
        lax.fori_loop(0, chunks, chunk, 0)
        plsc.subcore_barrier()
        pltpu.sync_copy(accum.at[pl.ds(row0, rpt)],
                        out_hbm.at[pl.ds(cid * n_pad + row0, rpt)])

    return pl.kernel(
        body, mesh=mesh,
        out_type=[jax.ShapeDtypeStruct((2 * n_pad, 128), jnp.float32)],
        scratch_types=[
            pltpu.VMEM((_KE,), jnp.int32),
            pltpu.VMEM((_KE, 128), jnp.float32),
            pltpu.VMEM((_ZB, 128), jnp.float32),
            pltpu.VMEM_SHARED((n_pad, 128), jnp.float32),
        ])


def _deg_sc(dst, n):
    (out,) = _deg_sc_make(n)(dst)
    n_pad = _npad(n)
    return out[:n] + out[n_pad:n_pad + n]


def _gather_sc_make(v, k):
    per_tile = k // _NT
    chunks = per_tile // _KE
    mesh = plsc.VectorSubcoreMesh(core_axis_name="c", subcore_axis_name="s")

    def body(table_hbm, idx_hbm, out_hbm, idx_v, rows, sem):
        cid = lax.axis_index("c")
        sid = lax.axis_index("s")
        wid = sid * 2 + cid

        def chunk(j, _):
            off = wid * per_tile + j * _KE
            pltpu.sync_copy(idx_hbm.at[pl.ds(off, _KE)], idx_v)
            pltpu.async_copy(table_hbm.at[idx_v], rows, sem).wait()
            pltpu.sync_copy(rows, out_hbm.at[pl.ds(off, _KE)])
            return 0
        lax.fori_loop(0, chunks, chunk, 0)

    return pl.kernel(
        body, mesh=mesh,
        out_type=[jax.ShapeDtypeStruct((k, 128), jnp.float32)],
        scratch_types=[
            pltpu.VMEM((_KE,), jnp.int32),
            pltpu.VMEM((_KE, 128), jnp.float32),
            pltpu.SemaphoreType.DMA,
        ])


def _gather_sc(table, idx):
    k0 = idx.shape[0]
    kq = _NT * _KE
    k_pad = ((k0 + kq - 1) // kq) * kq
    if k_pad != k0:
        idx = jnp.pad(idx, (0, k_pad - k0))
    (out,) = _gather_sc_make(table.shape[0], k_pad)(table, idx)
    return out[:k0]


def _dot(a, b):
    return jnp.dot(a, b, preferred_element_type=jnp.float32)


def _surf_body(chem_ref, geo_ref, feat_ref, wmp1, bmp1, g1, b1, wmp2, bmp2,
               wcat, bcat, wkp1, bkp1, wkp2, bkp2, out_ref):
    bm = chem_ref.shape[0]
    c = chem_ref[...].reshape(bm * 16, 15)
    h = _dot(c, wmp1[...]) + bmp1[...]
    h = h * g1[...] + b1[...]
    h = _lrelu(h, 0.1)
    h = _dot(h, wmp2[...]) + bmp2[...]
    cf = jnp.max(h.reshape(bm, 16, 21), axis=1)
    wc = wcat[...]
    es = (_dot(geo_ref[...], wc[0:10]) + _dot(cf, wc[10:31])
          + _dot(feat_ref[...], wc[31:32]) + bcat[...])
    t = _lrelu(_dot(es, wkp1[...]) + bkp1[...], 0.1)
    r = _dot(t, wkp2[...]) + bkp2[...]
    out_ref[...] = jnp.concatenate(
        [r, jnp.zeros((bm, 96), jnp.float32)], axis=1)


def _full_spec(a):
    r = len(a.shape)
    return pl.BlockSpec(a.shape, lambda *_, _r=r: (0,) * _r)


def _surface(chem, geo, features, p):
    bm = 800
    grid = (M // bm,)
    ws = [p['W_mp1'], p['b_mp1'], p['mp_bng'], p['mp_bnb'], p['W_mp2'], p['b_mp2'],
          p['W_cat'], p['b_cat'], p['W_kp1'], p['b_kp1'], p['W_kp2'], p['b_kp2']]
    return pl.pallas_call(
        _surf_body,
        grid=grid,
        in_specs=[pl.BlockSpec((bm, 16, 15), lambda i: (i, 0, 0)),
                  pl.BlockSpec((bm, 10), lambda i: (i, 0)),
                  pl.BlockSpec((bm, 1), lambda i: (i, 0))] + [_full_spec(w) for w in ws],
        out_specs=pl.BlockSpec((bm, 128), lambda i: (i, 0)),
        out_shape=jax.ShapeDtypeStruct((M, 128), jnp.float32),
    )(chem, geo, features, *ws)


def _kin_body(gs_ref, gd_ref, out_ref):
    gs = gs_ref[...]
    gd = gd_ref[...]
    rel = gd[:, 0:3] - gs[:, 0:3]
    r0 = rel[:, 0:1]
    r1 = rel[:, 1:2]
    r2 = rel[:, 2:3]
    rl0 = gd[:, 4:5] * r0 + gd[:, 5:6] * r1 + gd[:, 6:7] * r2
    rl1 = gd[:, 7:8] * r0 + gd[:, 8:9] * r1 + gd[:, 9:10] * r2
    rl2 = gd[:, 10:11] * r0 + gd[:, 11:12] * r1 + gd[:, 12:13] * r2
    rs = gd[:, 3:4] - gs[:, 3:4]
    out_ref[...] = jnp.concatenate([rl0, rl1, rl2, rs], axis=1)


def _kin_tc(gs, gd):
    ke = 3200
    return pl.pallas_call(
        _kin_body,
        grid=(E // ke,),
        in_specs=[pl.BlockSpec((ke, 128), lambda i: (i, 0)),
                  pl.BlockSpec((ke, 128), lambda i: (i, 0))],
        out_specs=pl.BlockSpec((ke, 4), lambda i: (i, 0)),
        out_shape=jax.ShapeDtypeStruct((E, 4), jnp.float32),
    )(gs, gd)


def _kern_body(kin_ref, wk1, bk1, wk2, out_ref):
    a = _dot(kin_ref[...], wk1[...]) + bk1[...]
    a = _lrelu(a, 0.1)
    out_ref[...] = _dot(a, wk2[...])


def _edge_kern(kin, wk1, bk1, wk2):
    ke = 3200
    return pl.pallas_call(
        _kern_body,
        grid=(E // ke,),
        in_specs=[pl.BlockSpec((ke, 4), lambda i: (i, 0)),
                  _full_spec(wk1), _full_spec(bk1), _full_spec(wk2)],
        out_specs=pl.BlockSpec((ke, W), lambda i: (i, 0)),
        out_shape=jax.ShapeDtypeStruct((E, W), jnp.float32),
    )(kin, wk1, bk1, wk2)


def _pre0_body(x_ref, fuse_ref, surf_g_ref, emb, wfus, bfus, g1, b1, inw, inb,
               g2, b2, gid, bid, idw, z_ref, id_ref):
    xi = x_ref[...]
    onehot = (xi == lax.broadcasted_iota(jnp.int32, (1, 21), 1)).astype(jnp.float32)
    h = _dot(onehot, emb[...])
    fuse = _dot(surf_g_ref[...][:, 0:32], wfus[...]) + bfus[...]
    del fuse_ref
    z = _lrelu(h * g1[...] + b1[...], 0.2)
    z = _dot(z, inw[...]) + inb[...]
    z = _lrelu(z * g2[...] + b2[...], 0.2)
    z_ref[...] = z + fuse
    id_ref[...] = _dot(_lrelu(h * gid[...] + bid[...], 0.1), idw[...])


def _pre0(x, surf_g, p):
    bn = 1000
    pre = 'b0'
    ws = [p['emb'], p['W_fus'], p['b_fus'],
          p[pre + '_in_bng1'], p[pre + '_in_bnb1'], p[pre + '_in_W'], p[pre + '_in_b'],
          p[pre + '_in_bng2'], p[pre + '_in_bnb2'],
          p[pre + '_id_bng'], p[pre + '_id_bnb'], p[pre + '_id_W']]
    dummy = jnp.zeros((N, 1), jnp.float32)
    return pl.pallas_call(
        _pre0_body,
        grid=(N // bn,),
        in_specs=[pl.BlockSpec((bn, 1), lambda i: (i, 0)),
                  pl.BlockSpec((bn, 1), lambda i: (i, 0)),
                  pl.BlockSpec((bn, 128), lambda i: (i, 0))] + [_full_spec(w) for w in ws],
        out_specs=[pl.BlockSpec((bn, W), lambda i: (i, 0)),
                   pl.BlockSpec((bn, CH), lambda i: (i, 0))],
        out_shape=[jax.ShapeDtypeStruct((N, W), jnp.float32),
                   jax.ShapeDtypeStruct((N, CH), jnp.float32)],
    )(x.reshape(N, 1), dummy, surf_g, *ws)


def _pre_body(h_ref, g1, b1, inw, inb, g2, b2, z_ref):
    z = _lrelu(h_ref[...] * g1[...] + b1[...], 0.2)
    z = _dot(z, inw[...]) + inb[...]
    z_ref[...] = _lrelu(z * g2[...] + b2[...], 0.2)


def _pre(h, p, i):
    n = h.shape[0]
    bn = min(n, 1024)
    pre = 'b%d' % i
    ws = [p[pre + '_in_bng1'], p[pre + '_in_bnb1'], p[pre + '_in_W'], p[pre + '_in_b'],
          p[pre + '_in_bng2'], p[pre + '_in_bnb2']]
    return pl.pallas_call(
        _pre_body,
        grid=(pl.cdiv(n, bn),),
        in_specs=[pl.BlockSpec((bn, CH), lambda i: (i, 0))] + [_full_spec(w) for w in ws],
        out_specs=pl.BlockSpec((bn, W), lambda i: (i, 0)),
        out_shape=jax.ShapeDtypeStruct((n, W), jnp.float32),
    )(h, *ws)


def _post_body(p0_ref, p1_ref, deg_ref, id_ref, wc, go, bo, outw, h_ref, *, pool):
    agg = (p0_ref[...] + p1_ref[...]) / jnp.maximum(deg_ref[...], 1.0)
    z = _dot(agg, wc[...])
    z = _lrelu(z * go[...] + bo[...], 0.1)
    hn = _dot(z, outw[...]) + id_ref[...]
    if pool:
        bn = hn.shape[0]
        hp = hn.reshape(bn // 2, 2, CH)
        h_ref[...] = (hp[:, 0, :] + hp[:, 1, :]) * 0.5
    else:
        h_ref[...] = hn


def _post(p0, p1, deg, ident, p, i, pool):
    n = p0.shape[0]
    bn = min(n, 1024)
    pre = 'b%d' % i
    ws = [p[pre + '_Wc'], p[pre + '_out_bng'], p[pre + '_out_bnb'], p[pre + '_out_W']]
    nout = n // 2 if pool else n
    bout = bn // 2 if pool else bn
    return pl.pallas_call(
        functools.partial(_post_body, pool=pool),
        grid=(pl.cdiv(n, bn),),
        in_specs=[pl.BlockSpec((bn, W), lambda i: (i, 0)),
                  pl.BlockSpec((bn, W), lambda i: (i, 0)),
                  pl.BlockSpec((bn, W), lambda i: (i, 0)),
                  pl.BlockSpec((bn, CH), lambda i: (i, 0))] + [_full_spec(w) for w in ws],
        out_specs=pl.BlockSpec((bout, CH), lambda i: (i, 0)),
        out_shape=jax.ShapeDtypeStruct((nout, CH), jnp.float32),
    )(p0, p1, deg, ident, *ws)


def _head_body(h_ref, batch_ref, g1, b1, w1, bb1, g2, b2, w2, bb2, out_ref):
    bid = batch_ref[...]
    mask = (bid == lax.broadcasted_iota(jnp.int32, (1, B), 1)).astype(jnp.float32)
    s = lax.dot_general(mask, h_ref[...], (((0,), (0,)), ((), ())),
                        preferred_element_type=jnp.float32)
    cnt = jnp.sum(mask, axis=0)[:, None]
    h = s / jnp.maximum(cnt, 1.0)
    z = _lrelu(h * g1[...] + b1[...], 0.2)
    z = _dot(z, w1[...]) + bb1[...]
    z = _lrelu(z * g2[...] + b2[...], 0.2)
    out_ref[...] = _dot(z, w2[...]) + bb2[...]


def _head(h, batch, p):
    n = h.shape[0]
    ws = [p['cls_bng1'], p['cls_bnb1'], p['cls_W1'], p['cls_b1'],
          p['cls_bng2'], p['cls_bnb2'], p['cls_W2'], p['cls_b2']]
    return pl.pallas_call(
        _head_body,
        in_specs=[pl.BlockSpec((n, CH), lambda: (0, 0)),
                  pl.BlockSpec((n, 1), lambda: (0, 0))] + [_full_spec(w) for w in ws],
        out_specs=pl.BlockSpec((B, 384), lambda: (0, 0)),
        out_shape=jax.ShapeDtypeStruct((B, 384), jnp.float32),
    )(h, batch.reshape(n, 1), *ws)


def kernel(x, pos, seq, ori, batch, edge_index, features, chem, geo,
           surf2struct, params):
    p = params
    out_surf = _surface(chem, geo, features, p)
    surf_g = _gather_sc(out_surf, surf2struct)

    g = jnp.concatenate([pos, seq, ori], axis=-1)
    src0, dst0 = edge_index[0], edge_index[1]

    h = None
    ident = None
    n = N
    for i in range(8):
        lvl = (i // 2)
        if i % 2 == 0:
            e_src = src0 // (1 << lvl)
            e_dst = dst0 // (1 << lvl)
            g_pad = jnp.pad(g, ((0, 0), (0, 128 - 13)))
            gs128 = _gather_sc(g_pad, e_src)
            gd128 = _gather_sc(g_pad, e_dst)
            kin = _kin_tc(gs128, gd128)
            if i == 0:
                deg = _deg_sc(e_dst, n)

        pre = 'b%d' % i
        if i == 0:
            z, ident = _pre0(x, surf_g, p)
        else:
            z = _pre(h, p, i)
            ident = h

        kern = _edge_kern(kin, p[pre + '_Wk1'], p[pre + '_bk1'], p[pre + '_Wk2'])
        p0, p1 = _conv_sc(z, kern, e_src, e_dst, n)

        pool = (i % 2 == 1) and i != 7
        h = _post(p0, p1, deg, ident, p, i, pool)
        if pool:
            g = (g[0::2] + g[1::2]) * 0.5
            deg = deg[0::2] + deg[1::2]
            batch = batch[0::2]
            n = n // 2

    return _head(h, batch, p)


# speedup vs baseline: 5.6512x; 1.1863x over previous
import functools

import jax
import jax.numpy as jnp
from jax import lax
from jax.experimental import pallas as pl
from jax.experimental.pallas import tpu as pltpu
from jax.experimental.pallas import tpu_sc as plsc

N, E, M, B = 10000, 320000, 20000, 8
CH, EMB, W, KC = 512, 16, 128, 24


def _lrelu(x, s):
    return jnp.where(x >= 0, x, s * x)


_KE = 80
_NT = 32
_ZB = 8


def _npad(n):
    return ((n + 127) // 128) * 128


_KC = 40


def _conv_sc_make(n):
    n_pad = _npad(n)
    rpt = n_pad // 16
    per_tile = E // _NT
    chunks = per_tile // _KC
    mesh = plsc.VectorSubcoreMesh(core_axis_name="c", subcore_axis_name="s")

    def body(z_hbm, kern_hbm, src_hbm, dst_hbm, out_hbm,
             idx_s0, idx_s1, idx_d0, idx_d1, rows0, rows1, kv0, kv1,
             msg0, msg1, zbuf, accum,
             semL0, semL1, semS0, semS1, semD0, semD1):
        cid = lax.axis_index("c")
        sid = lax.axis_index("s")
        wid = sid * 2 + cid
        zero16 = jnp.zeros((16,), jnp.float32)
        idx_s = (idx_s0, idx_s1)
        idx_d = (idx_d0, idx_d1)
        rows = (rows0, rows1)
        kv = (kv0, kv1)
        msg = (msg0, msg1)
        semL = (semL0, semL1)
        semS = (semS0, semS1)
        semD = (semD0, semD1)

        def zrow(r, _):
            for c in range(8):
                zbuf[r, pl.ds(c * 16, 16)] = zero16
            return 0
        lax.fori_loop(0, _ZB, zrow, 0)
        row0 = sid * rpt
        for b in range(rpt // _ZB):
            pltpu.sync_copy(zbuf, accum.at[pl.ds(row0 + b * _ZB, _ZB)])
        plsc.subcore_barrier()

        for b in (0, 1):
            off = wid * per_tile + b * _KC
            pltpu.sync_copy(src_hbm.at[pl.ds(off, _KC)], idx_s[b])
            pltpu.sync_copy(dst_hbm.at[pl.ds(off, _KC)], idx_d[b])
            pltpu.async_copy(z_hbm.at[idx_s[b]], rows[b], semL[b])
            pltpu.async_copy(kern_hbm.at[pl.ds(off, _KC)], kv[b], semL[b])

        def pair(k2, _):
            for b in (0, 1):
                k = 2 * k2 + b
                off = wid * per_tile + k * _KC
                pltpu.make_async_copy(z_hbm.at[idx_s[b]], rows[b],
                                      semL[b]).wait()
                pltpu.make_async_copy(kern_hbm.at[pl.ds(off, _KC)], kv[b],
                                      semL[b]).wait()

                @pl.when(k2 >= 1)
                def _():
                    pltpu.make_async_copy(kern_hbm.at[pl.ds(0, _KC)], msg[b],
                                          semS[b]).wait()
                    pltpu.async_copy(dst_hbm.at[pl.ds(off, _KC)], idx_d[b],
                                     semD[b])

                def mul(r, _, _b=b):
                    for c in range(8):
                        s = pl.ds(c * 16, 16)
                        msg[_b][r, s] = rows[_b][r, s] * kv[_b][r, s]
                    return 0
                lax.fori_loop(0, _KC, mul, 0)

                @pl.when(k2 >= 1)
                def _():
                    pltpu.make_async_copy(dst_hbm.at[pl.ds(off, _KC)],
                                          idx_d[b], semD[b]).wait()
                pltpu.async_copy(msg[b], accum.at[idx_d[b]], semS[b],
                                 add=True)

                @pl.when(k2 < chunks // 2 - 1)
                def _():
                    off2 = off + 2 * _KC
                    pltpu.sync_copy(src_hbm.at[pl.ds(off2, _KC)], idx_s[b])
                    pltpu.async_copy(z_hbm.at[idx_s[b]], rows[b], semL[b])
                    pltpu.async_copy(kern_hbm.at[pl.ds(off2, _KC)], kv[b],
                                     semL[b])
            return 0
        lax.fori_loop(0, chunks // 2, pair, 0)
        for b in (0, 1):
            pltpu.make_async_copy(kern_hbm.at[pl.ds(0, _KC)], msg[b],
                                  semS[b]).wait()
        plsc.subcore_barrier()
        pltpu.sync_copy(accum.at[pl.ds(row0, rpt)],
                        out_hbm.at[pl.ds(cid * n_pad + row0, rpt)])

    return pl.kernel(
        body, mesh=mesh,
        out_type=[jax.ShapeDtypeStruct((2 * n_pad, 128), jnp.float32)],
        scratch_types=[
            pltpu.VMEM((_KC,), jnp.int32),
            pltpu.VMEM((_KC,), jnp.int32),
            pltpu.VMEM((_KC,), jnp.int32),
            pltpu.VMEM((_KC,), jnp.int32),
            pltpu.VMEM((_KC, 128), jnp.float32),
            pltpu.VMEM((_KC, 128), jnp.float32),
            pltpu.VMEM((_KC, 128), jnp.float32),
            pltpu.VMEM((_KC, 128), jnp.float32),
            pltpu.VMEM((_KC, 128), jnp.float32),
            pltpu.VMEM((_KC, 128), jnp.float32),
            pltpu.VMEM((_ZB, 128), jnp.float32),
            pltpu.VMEM_SHARED((n_pad, 128), jnp.float32),
            pltpu.SemaphoreType.DMA,
            pltpu.SemaphoreType.DMA,
            pltpu.SemaphoreType.DMA,
            pltpu.SemaphoreType.DMA,
            pltpu.SemaphoreType.DMA,
            pltpu.SemaphoreType.DMA,
        ])


def _conv_sc(z, kern, src, dst, n):
    (out,) = _conv_sc_make(n)(z, kern, src, dst)
    n_pad = _npad(n)
    return out[:n], out[n_pad:n_pad + n]


def _deg_sc_make(n):
    n_pad = _npad(n)
    rpt = n_pad // 16
    per_tile = E // _NT
    chunks = per_tile // _KE
    mesh = plsc.VectorSubcoreMesh(core_axis_name="c", subcore_axis_name="s")

    def body(dst_hbm, out_hbm, idx_d, ones_v, zbuf, accum):
        cid = lax.axis_index("c")
        sid = lax.axis_index("s")
        wid = sid * 2 + cid
        zero16 = jnp.zeros((16,), jnp.float32)
        one16 = jnp.ones((16,), jnp.float32)

        def orow(r, _):
            for c in range(8):
                ones_v[r, pl.ds(c * 16, 16)] = one16
            return 0
        lax.fori_loop(0, _KE, orow, 0)

        def zrow(r, _):
            for c in range(8):
                zbuf[r, pl.ds(c * 16, 16)] = zero16
            return 0
        lax.fori_loop(0, _ZB, zrow, 0)
        row0 = sid * rpt
        for b in range(rpt // _ZB):
            pltpu.sync_copy(zbuf, accum.at[pl.ds(row0 + b * _ZB, _ZB)])
        plsc.subcore_barrier()

        def chunk(k, _):
            off = wid * per_tile + k * _KE
            pltpu.sync_copy(dst_hbm.at[pl.ds(off, _KE)], idx_d)
            pltpu.sync_copy(ones_v, accum.at[idx_d], add=True)
            return 0
        lax.fori_loop(0, chunks, chunk, 0)
        plsc.subcore_barrier()
        pltpu.sync_copy(accum.at[pl.ds(row0, rpt)],
                        out_hbm.at[pl.ds(cid * n_pad + row0, rpt)])

    return pl.kernel(
        body, mesh=mesh,
        out_type=[jax.ShapeDtypeStruct((2 * n_pad, 128), jnp.float32)],
        scratch_types=[
            pltpu.VMEM((_KE,), jnp.int32),
            pltpu.VMEM((_KE, 128), jnp.float32),
            pltpu.VMEM((_ZB, 128), jnp.float32),
            pltpu.VMEM_SHARED((n_pad, 128), jnp.float32),
        ])


def _deg_sc(dst, n):
    (out,) = _deg_sc_make(n)(dst)
    n_pad = _npad(n)
    return out[:n] + out[n_pad:n_pad + n]


def _gather_sc_make(v, k):
    per_tile = k // _NT
    chunks = per_tile // _KE
    mesh = plsc.VectorSubcoreMesh(core_axis_name="c", subcore_axis_name="s")

    def body(table_hbm, idx_hbm, out_hbm, idx_v, rows, sem):
        cid = lax.axis_index("c")
        sid = lax.axis_index("s")
        wid = sid * 2 + cid

        def chunk(j, _):
            off = wid * per_tile + j * _KE
            pltpu.sync_copy(idx_hbm.at[pl.ds(off, _KE)], idx_v)
            pltpu.async_copy(table_hbm.at[idx_v], rows, sem).wait()
            pltpu.sync_copy(rows, out_hbm.at[pl.ds(off, _KE)])
            return 0
        lax.fori_loop(0, chunks, chunk, 0)

    return pl.kernel(
        body, mesh=mesh,
        out_type=[jax.ShapeDtypeStruct((k, 128), jnp.float32)],
        scratch_types=[
            pltpu.VMEM((_KE,), jnp.int32),
            pltpu.VMEM((_KE, 128), jnp.float32),
            pltpu.SemaphoreType.DMA,
        ])


def _gather_sc(table, idx):
    k0 = idx.shape[0]
    kq = _NT * _KE
    k_pad = ((k0 + kq - 1) // kq) * kq
    if k_pad != k0:
        idx = jnp.pad(idx, (0, k_pad - k0))
    (out,) = _gather_sc_make(table.shape[0], k_pad)(table, idx)
    return out[:k0]


def _dot(a, b):
    return jnp.dot(a, b, preferred_element_type=jnp.float32)


def _surf_body(chem_ref, geo_ref, feat_ref, wmp1, bmp1, g1, b1, wmp2, bmp2,
               wcat, bcat, wkp1, bkp1, wkp2, bkp2, out_ref):
    bm = chem_ref.shape[0]
    c = chem_ref[...].reshape(bm * 16, 15)
    h = _dot(c, wmp1[...]) + bmp1[...]
    h = h * g1[...] + b1[...]
    h = _lrelu(h, 0.1)
    h = _dot(h, wmp2[...]) + bmp2[...]
    cf = jnp.max(h.reshape(bm, 16, 21), axis=1)
    wc = wcat[...]
    es = (_dot(geo_ref[...], wc[0:10]) + _dot(cf, wc[10:31])
          + _dot(feat_ref[...], wc[31:32]) + bcat[...])
    t = _lrelu(_dot(es, wkp1[...]) + bkp1[...], 0.1)
    r = _dot(t, wkp2[...]) + bkp2[...]
    out_ref[...] = jnp.concatenate(
        [r, jnp.zeros((bm, 96), jnp.float32)], axis=1)


def _full_spec(a):
    r = len(a.shape)
    return pl.BlockSpec(a.shape, lambda *_, _r=r: (0,) * _r)


def _surface(chem, geo, features, p):
    bm = 800
    grid = (M // bm,)
    ws = [p['W_mp1'], p['b_mp1'], p['mp_bng'], p['mp_bnb'], p['W_mp2'], p['b_mp2'],
          p['W_cat'], p['b_cat'], p['W_kp1'], p['b_kp1'], p['W_kp2'], p['b_kp2']]
    return pl.pallas_call(
        _surf_body,
        grid=grid,
        in_specs=[pl.BlockSpec((bm, 16, 15), lambda i: (i, 0, 0)),
                  pl.BlockSpec((bm, 10), lambda i: (i, 0)),
                  pl.BlockSpec((bm, 1), lambda i: (i, 0))] + [_full_spec(w) for w in ws],
        out_specs=pl.BlockSpec((bm, 128), lambda i: (i, 0)),
        out_shape=jax.ShapeDtypeStruct((M, 128), jnp.float32),
    )(chem, geo, features, *ws)


def _kin_body(gs_ref, gd_ref, out_ref):
    gs = gs_ref[...]
    gd = gd_ref[...]
    rel = gd[:, 0:3] - gs[:, 0:3]
    r0 = rel[:, 0:1]
    r1 = rel[:, 1:2]
    r2 = rel[:, 2:3]
    rl0 = gd[:, 4:5] * r0 + gd[:, 5:6] * r1 + gd[:, 6:7] * r2
    rl1 = gd[:, 7:8] * r0 + gd[:, 8:9] * r1 + gd[:, 9:10] * r2
    rl2 = gd[:, 10:11] * r0 + gd[:, 11:12] * r1 + gd[:, 12:13] * r2
    rs = gd[:, 3:4] - gs[:, 3:4]
    out_ref[...] = jnp.concatenate([rl0, rl1, rl2, rs], axis=1)


def _kin_tc(gs, gd):
    ke = 3200
    return pl.pallas_call(
        _kin_body,
        grid=(E // ke,),
        in_specs=[pl.BlockSpec((ke, 128), lambda i: (i, 0)),
                  pl.BlockSpec((ke, 128), lambda i: (i, 0))],
        out_specs=pl.BlockSpec((ke, 4), lambda i: (i, 0)),
        out_shape=jax.ShapeDtypeStruct((E, 4), jnp.float32),
    )(gs, gd)


def _kern_body(kin_ref, wk1, bk1, wk2, out_ref):
    a = _dot(kin_ref[...], wk1[...]) + bk1[...]
    a = _lrelu(a, 0.1)
    out_ref[...] = _dot(a, wk2[...])


def _edge_kern(kin, wk1, bk1, wk2):
    ke = 3200
    return pl.pallas_call(
        _kern_body,
        grid=(E // ke,),
        in_specs=[pl.BlockSpec((ke, 4), lambda i: (i, 0)),
                  _full_spec(wk1), _full_spec(bk1), _full_spec(wk2)],
        out_specs=pl.BlockSpec((ke, W), lambda i: (i, 0)),
        out_shape=jax.ShapeDtypeStruct((E, W), jnp.float32),
    )(kin, wk1, bk1, wk2)


def _pre0_body(x_ref, fuse_ref, surf_g_ref, emb, wfus, bfus, g1, b1, inw, inb,
               g2, b2, gid, bid, idw, z_ref, id_ref):
    xi = x_ref[...]
    onehot = (xi == lax.broadcasted_iota(jnp.int32, (1, 21), 1)).astype(jnp.float32)
    h = _dot(onehot, emb[...])
    fuse = _dot(surf_g_ref[...][:, 0:32], wfus[...]) + bfus[...]
    del fuse_ref
    z = _lrelu(h * g1[...] + b1[...], 0.2)
    z = _dot(z, inw[...]) + inb[...]
    z = _lrelu(z * g2[...] + b2[...], 0.2)
    z_ref[...] = z + fuse
    id_ref[...] = _dot(_lrelu(h * gid[...] + bid[...], 0.1), idw[...])


def _pre0(x, surf_g, p):
    bn = 1000
    pre = 'b0'
    ws = [p['emb'], p['W_fus'], p['b_fus'],
          p[pre + '_in_bng1'], p[pre + '_in_bnb1'], p[pre + '_in_W'], p[pre + '_in_b'],
          p[pre + '_in_bng2'], p[pre + '_in_bnb2'],
          p[pre + '_id_bng'], p[pre + '_id_bnb'], p[pre + '_id_W']]
    dummy = jnp.zeros((N, 1), jnp.float32)
    return pl.pallas_call(
        _pre0_body,
        grid=(N // bn,),
        in_specs=[pl.BlockSpec((bn, 1), lambda i: (i, 0)),
                  pl.BlockSpec((bn, 1), lambda i: (i, 0)),
                  pl.BlockSpec((bn, 128), lambda i: (i, 0))] + [_full_spec(w) for w in ws],
        out_specs=[pl.BlockSpec((bn, W), lambda i: (i, 0)),
                   pl.BlockSpec((bn, CH), lambda i: (i, 0))],
        out_shape=[jax.ShapeDtypeStruct((N, W), jnp.float32),
                   jax.ShapeDtypeStruct((N, CH), jnp.float32)],
    )(x.reshape(N, 1), dummy, surf_g, *ws)


def _pre_body(h_ref, g1, b1, inw, inb, g2, b2, z_ref):
    z = _lrelu(h_ref[...] * g1[...] + b1[...], 0.2)
    z = _dot(z, inw[...]) + inb[...]
    z_ref[...] = _lrelu(z * g2[...] + b2[...], 0.2)


def _pre(h, p, i):
    n = h.shape[0]
    bn = min(n, 1024)
    pre = 'b%d' % i
    ws = [p[pre + '_in_bng1'], p[pre + '_in_bnb1'], p[pre + '_in_W'], p[pre + '_in_b'],
          p[pre + '_in_bng2'], p[pre + '_in_bnb2']]
    return pl.pallas_call(
        _pre_body,
        grid=(pl.cdiv(n, bn),),
        in_specs=[pl.BlockSpec((bn, CH), lambda i: (i, 0))] + [_full_spec(w) for w in ws],
        out_specs=pl.BlockSpec((bn, W), lambda i: (i, 0)),
        out_shape=jax.ShapeDtypeStruct((n, W), jnp.float32),
    )(h, *ws)


def _post_body(p0_ref, p1_ref, deg_ref, id_ref, wc, go, bo, outw, h_ref, *, pool):
    agg = (p0_ref[...] + p1_ref[...]) / jnp.maximum(deg_ref[...], 1.0)
    z = _dot(agg, wc[...])
    z = _lrelu(z * go[...] + bo[...], 0.1)
    hn = _dot(z, outw[...]) + id_ref[...]
    if pool:
        bn = hn.shape[0]
        hp = hn.reshape(bn // 2, 2, CH)
        h_ref[...] = (hp[:, 0, :] + hp[:, 1, :]) * 0.5
    else:
        h_ref[...] = hn


def _post(p0, p1, deg, ident, p, i, pool):
    n = p0.shape[0]
    bn = min(n, 1024)
    pre = 'b%d' % i
    ws = [p[pre + '_Wc'], p[pre + '_out_bng'], p[pre + '_out_bnb'], p[pre + '_out_W']]
    nout = n // 2 if pool else n
    bout = bn // 2 if pool else bn
    return pl.pallas_call(
        functools.partial(_post_body, pool=pool),
        grid=(pl.cdiv(n, bn),),
        in_specs=[pl.BlockSpec((bn, W), lambda i: (i, 0)),
                  pl.BlockSpec((bn, W), lambda i: (i, 0)),
                  pl.BlockSpec((bn, W), lambda i: (i, 0)),
                  pl.BlockSpec((bn, CH), lambda i: (i, 0))] + [_full_spec(w) for w in ws],
        out_specs=pl.BlockSpec((bout, CH), lambda i: (i, 0)),
        out_shape=jax.ShapeDtypeStruct((nout, CH), jnp.float32),
    )(p0, p1, deg, ident, *ws)


def _head_body(h_ref, batch_ref, g1, b1, w1, bb1, g2, b2, w2, bb2, out_ref):
    bid = batch_ref[...]
    mask = (bid == lax.broadcasted_iota(jnp.int32, (1, B), 1)).astype(jnp.float32)
    s = lax.dot_general(mask, h_ref[...], (((0,), (0,)), ((), ())),
                        preferred_element_type=jnp.float32)
    cnt = jnp.sum(mask, axis=0)[:, None]
    h = s / jnp.maximum(cnt, 1.0)
    z = _lrelu(h * g1[...] + b1[...], 0.2)
    z = _dot(z, w1[...]) + bb1[...]
    z = _lrelu(z * g2[...] + b2[...], 0.2)
    out_ref[...] = _dot(z, w2[...]) + bb2[...]


def _head(h, batch, p):
    n = h.shape[0]
    ws = [p['cls_bng1'], p['cls_bnb1'], p['cls_W1'], p['cls_b1'],
          p['cls_bng2'], p['cls_bnb2'], p['cls_W2'], p['cls_b2']]
    return pl.pallas_call(
        _head_body,
        in_specs=[pl.BlockSpec((n, CH), lambda: (0, 0)),
                  pl.BlockSpec((n, 1), lambda: (0, 0))] + [_full_spec(w) for w in ws],
        out_specs=pl.BlockSpec((B, 384), lambda: (0, 0)),
        out_shape=jax.ShapeDtypeStruct((B, 384), jnp.float32),
    )(h, batch.reshape(n, 1), *ws)


def kernel(x, pos, seq, ori, batch, edge_index, features, chem, geo,
           surf2struct, params):
    p = params
    out_surf = _surface(chem, geo, features, p)
    surf_g = _gather_sc(out_surf, surf2struct)

    g = jnp.concatenate([pos, seq, ori], axis=-1)
    src0, dst0 = edge_index[0], edge_index[1]

    h = None
    ident = None
    n = N
    for i in range(8):
        lvl = (i // 2)
        if i % 2 == 0:
            e_src = src0 // (1 << lvl)
            e_dst = dst0 // (1 << lvl)
            g_pad = jnp.pad(g, ((0, 0), (0, 128 - 13)))
            gs128 = _gather_sc(g_pad, e_src)
            gd128 = _gather_sc(g_pad, e_dst)
            kin = _kin_tc(gs128, gd128)
            if i == 0:
                deg = _deg_sc(e_dst, n)

        pre = 'b%d' % i
        if i == 0:
            z, ident = _pre0(x, surf_g, p)
        else:
            z = _pre(h, p, i)
            ident = h

        kern = _edge_kern(kin, p[pre + '_Wk1'], p[pre + '_bk1'], p[pre + '_Wk2'])
        p0, p1 = _conv_sc(z, kern, e_src, e_dst, n)

        pool = (i % 2 == 1) and i != 7
        h = _post(p0, p1, deg, ident, p, i, pool)
        if pool:
            g = (g[0::2] + g[1::2]) * 0.5
            deg = deg[0::2] + deg[1::2]
            batch = batch[0::2]
            n = n // 2

    return _head(h, batch, p)


# speedup vs baseline: 6.0920x; 1.0780x over previous
import functools

import jax
import jax.numpy as jnp
from jax import lax
from jax.experimental import pallas as pl
from jax.experimental.pallas import tpu as pltpu
from jax.experimental.pallas import tpu_sc as plsc

N, E, M, B = 10000, 320000, 20000, 8
CH, EMB, W, KC = 512, 16, 128, 24


def _lrelu(x, s):
    return jnp.where(x >= 0, x, s * x)


_KE = 80
_NT = 32
_ZB = 8


def _npad(n):
    return ((n + 127) // 128) * 128


_KC = 40


def _conv_sc_make(n):
    n_pad = _npad(n)
    rpt = n_pad // 16
    per_tile = E // _NT
    chunks = per_tile // _KC
    mesh = plsc.VectorSubcoreMesh(core_axis_name="c", subcore_axis_name="s")

    def body(z_hbm, kern_hbm, src_hbm, dst_hbm, out_hbm,
             idx_s0, idx_s1, idx_d0, idx_d1, rows0, rows1, kv0, kv1,
             msg0, msg1, zbuf, accum,
             semL0, semL1, semS0, semS1, semD0, semD1):
        cid = lax.axis_index("c")
        sid = lax.axis_index("s")
        wid = sid * 2 + cid
        zero16 = jnp.zeros((16,), jnp.float32)
        idx_s = (idx_s0, idx_s1)
        idx_d = (idx_d0, idx_d1)
        rows = (rows0, rows1)
        kv = (kv0, kv1)
        msg = (msg0, msg1)
        semL = (semL0, semL1)
        semS = (semS0, semS1)
        semD = (semD0, semD1)

        def zrow(r, _):
            for c in range(8):
                zbuf[r, pl.ds(c * 16, 16)] = zero16
            return 0
        lax.fori_loop(0, _ZB, zrow, 0)
        row0 = sid * rpt
        for b in range(rpt // _ZB):
            pltpu.sync_copy(zbuf, accum.at[pl.ds(row0 + b * _ZB, _ZB)])
        plsc.subcore_barrier()

        for b in (0, 1):
            off = wid * per_tile + b * _KC
            pltpu.sync_copy(src_hbm.at[pl.ds(off, _KC)], idx_s[b])
            pltpu.sync_copy(dst_hbm.at[pl.ds(off, _KC)], idx_d[b])
            pltpu.async_copy(z_hbm.at[idx_s[b]], rows[b], semL[b])
            pltpu.async_copy(kern_hbm.at[pl.ds(off, _KC)], kv[b], semL[b])

        def pair(k2, _):
            for b in (0, 1):
                k = 2 * k2 + b
                off = wid * per_tile + k * _KC
                pltpu.make_async_copy(z_hbm.at[idx_s[b]], rows[b],
                                      semL[b]).wait()
                pltpu.make_async_copy(kern_hbm.at[pl.ds(off, _KC)], kv[b],
                                      semL[b]).wait()

                @pl.when(k2 >= 1)
                def _():
                    pltpu.make_async_copy(kern_hbm.at[pl.ds(0, _KC)], msg[b],
                                          semS[b]).wait()
                    pltpu.async_copy(dst_hbm.at[pl.ds(off, _KC)], idx_d[b],
                                     semD[b])

                def mul(r, _, _b=b):
                    for c in range(8):
                        s = pl.ds(c * 16, 16)
                        msg[_b][r, s] = rows[_b][r, s] * kv[_b][r, s]
                    return 0
                lax.fori_loop(0, _KC, mul, 0)

                @pl.when(k2 >= 1)
                def _():
                    pltpu.make_async_copy(dst_hbm.at[pl.ds(off, _KC)],
                                          idx_d[b], semD[b]).wait()
                pltpu.async_copy(msg[b], accum.at[idx_d[b]], semS[b],
                                 add=True)

                @pl.when(k2 < chunks // 2 - 1)
                def _():
                    off2 = off + 2 * _KC
                    pltpu.sync_copy(src_hbm.at[pl.ds(off2, _KC)], idx_s[b])
                    pltpu.async_copy(z_hbm.at[idx_s[b]], rows[b], semL[b])
                    pltpu.async_copy(kern_hbm.at[pl.ds(off2, _KC)], kv[b],
                                     semL[b])
            return 0
        lax.fori_loop(0, chunks // 2, pair, 0)
        for b in (0, 1):
            pltpu.make_async_copy(kern_hbm.at[pl.ds(0, _KC)], msg[b],
                                  semS[b]).wait()
        plsc.subcore_barrier()
        pltpu.sync_copy(accum.at[pl.ds(row0, rpt)],
                        out_hbm.at[pl.ds(cid * n_pad + row0, rpt)])

    return pl.kernel(
        body, mesh=mesh,
        out_type=[jax.ShapeDtypeStruct((2 * n_pad, 128), jnp.float32)],
        scratch_types=[
            pltpu.VMEM((_KC,), jnp.int32),
            pltpu.VMEM((_KC,), jnp.int32),
            pltpu.VMEM((_KC,), jnp.int32),
            pltpu.VMEM((_KC,), jnp.int32),
            pltpu.VMEM((_KC, 128), jnp.float32),
            pltpu.VMEM((_KC, 128), jnp.float32),
            pltpu.VMEM((_KC, 128), jnp.float32),
            pltpu.VMEM((_KC, 128), jnp.float32),
            pltpu.VMEM((_KC, 128), jnp.float32),
            pltpu.VMEM((_KC, 128), jnp.float32),
            pltpu.VMEM((_ZB, 128), jnp.float32),
            pltpu.VMEM_SHARED((n_pad, 128), jnp.float32),
            pltpu.SemaphoreType.DMA,
            pltpu.SemaphoreType.DMA,
            pltpu.SemaphoreType.DMA,
            pltpu.SemaphoreType.DMA,
            pltpu.SemaphoreType.DMA,
            pltpu.SemaphoreType.DMA,
        ])


def _conv_sc(z, kern, src, dst, n):
    (out,) = _conv_sc_make(n)(z, kern, src, dst)
    n_pad = _npad(n)
    return out[:n], out[n_pad:n_pad + n]


def _deg_sc_make(n):
    n_pad = _npad(n)
    rpt = n_pad // 16
    per_tile = E // _NT
    chunks = per_tile // _KE
    mesh = plsc.VectorSubcoreMesh(core_axis_name="c", subcore_axis_name="s")

    def body(dst_hbm, out_hbm, idx_d, ones_v, zbuf, accum):
        cid = lax.axis_index("c")
        sid = lax.axis_index("s")
        wid = sid * 2 + cid
        zero16 = jnp.zeros((16,), jnp.float32)
        one16 = jnp.ones((16,), jnp.float32)

        def orow(r, _):
            for c in range(8):
                ones_v[r, pl.ds(c * 16, 16)] = one16
            return 0
        lax.fori_loop(0, _KE, orow, 0)

        def zrow(r, _):
            for c in range(8):
                zbuf[r, pl.ds(c * 16, 16)] = zero16
            return 0
        lax.fori_loop(0, _ZB, zrow, 0)
        row0 = sid * rpt
        for b in range(rpt // _ZB):
            pltpu.sync_copy(zbuf, accum.at[pl.ds(row0 + b * _ZB, _ZB)])
        plsc.subcore_barrier()

        def chunk(k, _):
            off = wid * per_tile + k * _KE
            pltpu.sync_copy(dst_hbm.at[pl.ds(off, _KE)], idx_d)
            pltpu.sync_copy(ones_v, accum.at[idx_d], add=True)
            return 0
        lax.fori_loop(0, chunks, chunk, 0)
        plsc.subcore_barrier()
        pltpu.sync_copy(accum.at[pl.ds(row0, rpt)],
                        out_hbm.at[pl.ds(cid * n_pad + row0, rpt)])

    return pl.kernel(
        body, mesh=mesh,
        out_type=[jax.ShapeDtypeStruct((2 * n_pad, 128), jnp.float32)],
        scratch_types=[
            pltpu.VMEM((_KE,), jnp.int32),
            pltpu.VMEM((_KE, 128), jnp.float32),
            pltpu.VMEM((_ZB, 128), jnp.float32),
            pltpu.VMEM_SHARED((n_pad, 128), jnp.float32),
        ])


def _deg_sc(dst, n):
    (out,) = _deg_sc_make(n)(dst)
    n_pad = _npad(n)
    return out[:n] + out[n_pad:n_pad + n]


def _gather_sc_make(v, k):
    per_tile = k // _NT
    chunks = per_tile // _KE
    mesh = plsc.VectorSubcoreMesh(core_axis_name="c", subcore_axis_name="s")

    def body(table_hbm, idx_hbm, out_hbm, idx_v, rows, sem):
        cid = lax.axis_index("c")
        sid = lax.axis_index("s")
        wid = sid * 2 + cid

        def chunk(j, _):
            off = wid * per_tile + j * _KE
            pltpu.sync_copy(idx_hbm.at[pl.ds(off, _KE)], idx_v)
            pltpu.async_copy(table_hbm.at[idx_v], rows, sem).wait()
            pltpu.sync_copy(rows, out_hbm.at[pl.ds(off, _KE)])
            return 0
        lax.fori_loop(0, chunks, chunk, 0)

    return pl.kernel(
        body, mesh=mesh,
        out_type=[jax.ShapeDtypeStruct((k, 128), jnp.float32)],
        scratch_types=[
            pltpu.VMEM((_KE,), jnp.int32),
            pltpu.VMEM((_KE, 128), jnp.float32),
            pltpu.SemaphoreType.DMA,
        ])


def _geo_sc_make(v):
    per_tile = E // _NT
    chunks = per_tile // _KE
    mesh = plsc.VectorSubcoreMesh(core_axis_name="c", subcore_axis_name="s")

    def body(table_hbm, src_hbm, dst_hbm, gs_hbm, gd_hbm,
             idx_s, idx_d, rows_s, rows_d, semA, semB):
        cid = lax.axis_index("c")
        sid = lax.axis_index("s")
        wid = sid * 2 + cid

        def chunk(j, _):
            off = wid * per_tile + j * _KE
            pltpu.sync_copy(src_hbm.at[pl.ds(off, _KE)], idx_s)
            pltpu.sync_copy(dst_hbm.at[pl.ds(off, _KE)], idx_d)
            pltpu.async_copy(table_hbm.at[idx_s], rows_s, semA)
            pltpu.async_copy(table_hbm.at[idx_d], rows_d, semB)
            pltpu.make_async_copy(table_hbm.at[idx_s], rows_s, semA).wait()
            pltpu.make_async_copy(table_hbm.at[idx_d], rows_d, semB).wait()
            pltpu.sync_copy(rows_s, gs_hbm.at[pl.ds(off, _KE)])
            pltpu.sync_copy(rows_d, gd_hbm.at[pl.ds(off, _KE)])
            return 0
        lax.fori_loop(0, chunks, chunk, 0)

    return pl.kernel(
        body, mesh=mesh,
        out_type=[jax.ShapeDtypeStruct((E, 128), jnp.float32),
                  jax.ShapeDtypeStruct((E, 128), jnp.float32)],
        scratch_types=[
            pltpu.VMEM((_KE,), jnp.int32),
            pltpu.VMEM((_KE,), jnp.int32),
            pltpu.VMEM((_KE, 128), jnp.float32),
            pltpu.VMEM((_KE, 128), jnp.float32),
            pltpu.SemaphoreType.DMA,
            pltpu.SemaphoreType.DMA,
        ])


def _geo_sc(table, src, dst):
    return _geo_sc_make(table.shape[0])(table, src, dst)


def _gather_sc(table, idx):
    k0 = idx.shape[0]
    kq = _NT * _KE
    k_pad = ((k0 + kq - 1) // kq) * kq
    if k_pad != k0:
        idx = jnp.pad(idx, (0, k_pad - k0))
    (out,) = _gather_sc_make(table.shape[0], k_pad)(table, idx)
    return out[:k0]


def _dot(a, b):
    return jnp.dot(a, b, preferred_element_type=jnp.float32)


def _surf_body(chem_ref, geo_ref, feat_ref, wmp1, bmp1, g1, b1, wmp2, bmp2,
               wcat, bcat, wkp1, bkp1, wkp2, bkp2, out_ref):
    bm = chem_ref.shape[0]
    c = chem_ref[...].reshape(bm * 16, 15)
    h = _dot(c, wmp1[...]) + bmp1[...]
    h = h * g1[...] + b1[...]
    h = _lrelu(h, 0.1)
    h = _dot(h, wmp2[...]) + bmp2[...]
    cf = jnp.max(h.reshape(bm, 16, 21), axis=1)
    wc = wcat[...]
    es = (_dot(geo_ref[...], wc[0:10]) + _dot(cf, wc[10:31])
          + _dot(feat_ref[...], wc[31:32]) + bcat[...])
    t = _lrelu(_dot(es, wkp1[...]) + bkp1[...], 0.1)
    r = _dot(t, wkp2[...]) + bkp2[...]
    out_ref[...] = jnp.concatenate(
        [r, jnp.zeros((bm, 96), jnp.float32)], axis=1)


def _full_spec(a):
    r = len(a.shape)
    return pl.BlockSpec(a.shape, lambda *_, _r=r: (0,) * _r)


def _surface(chem, geo, features, p):
    bm = 800
    grid = (M // bm,)
    ws = [p['W_mp1'], p['b_mp1'], p['mp_bng'], p['mp_bnb'], p['W_mp2'], p['b_mp2'],
          p['W_cat'], p['b_cat'], p['W_kp1'], p['b_kp1'], p['W_kp2'], p['b_kp2']]
    return pl.pallas_call(
        _surf_body,
        grid=grid,
        in_specs=[pl.BlockSpec((bm, 16, 15), lambda i: (i, 0, 0)),
                  pl.BlockSpec((bm, 10), lambda i: (i, 0)),
                  pl.BlockSpec((bm, 1), lambda i: (i, 0))] + [_full_spec(w) for w in ws],
        out_specs=pl.BlockSpec((bm, 128), lambda i: (i, 0)),
        out_shape=jax.ShapeDtypeStruct((M, 128), jnp.float32),
    )(chem, geo, features, *ws)


def _kin_body(gs_ref, gd_ref, out_ref):
    gs = gs_ref[...]
    gd = gd_ref[...]
    rel = gd[:, 0:3] - gs[:, 0:3]
    r0 = rel[:, 0:1]
    r1 = rel[:, 1:2]
    r2 = rel[:, 2:3]
    rl0 = gd[:, 4:5] * r0 + gd[:, 5:6] * r1 + gd[:, 6:7] * r2
    rl1 = gd[:, 7:8] * r0 + gd[:, 8:9] * r1 + gd[:, 9:10] * r2
    rl2 = gd[:, 10:11] * r0 + gd[:, 11:12] * r1 + gd[:, 12:13] * r2
    rs = gd[:, 3:4] - gs[:, 3:4]
    out_ref[...] = jnp.concatenate([rl0, rl1, rl2, rs], axis=1)


def _kin_tc(gs, gd):
    ke = 3200
    return pl.pallas_call(
        _kin_body,
        grid=(E // ke,),
        in_specs=[pl.BlockSpec((ke, 128), lambda i: (i, 0)),
                  pl.BlockSpec((ke, 128), lambda i: (i, 0))],
        out_specs=pl.BlockSpec((ke, 4), lambda i: (i, 0)),
        out_shape=jax.ShapeDtypeStruct((E, 4), jnp.float32),
    )(gs, gd)


def _kern_body(kin_ref, wk1, bk1, wk2, out_ref):
    a = _dot(kin_ref[...], wk1[...]) + bk1[...]
    a = _lrelu(a, 0.1)
    out_ref[...] = _dot(a, wk2[...])


def _edge_kern(kin, wk1, bk1, wk2):
    ke = 3200
    return pl.pallas_call(
        _kern_body,
        grid=(E // ke,),
        in_specs=[pl.BlockSpec((ke, 4), lambda i: (i, 0)),
                  _full_spec(wk1), _full_spec(bk1), _full_spec(wk2)],
        out_specs=pl.BlockSpec((ke, W), lambda i: (i, 0)),
        out_shape=jax.ShapeDtypeStruct((E, W), jnp.float32),
    )(kin, wk1, bk1, wk2)


def _pre0_body(x_ref, fuse_ref, surf_g_ref, emb, wfus, bfus, g1, b1, inw, inb,
               g2, b2, gid, bid, idw, z_ref, id_ref):
    xi = x_ref[...]
    onehot = (xi == lax.broadcasted_iota(jnp.int32, (1, 21), 1)).astype(jnp.float32)
    h = _dot(onehot, emb[...])
    fuse = _dot(surf_g_ref[...][:, 0:32], wfus[...]) + bfus[...]
    del fuse_ref
    z = _lrelu(h * g1[...] + b1[...], 0.2)
    z = _dot(z, inw[...]) + inb[...]
    z = _lrelu(z * g2[...] + b2[...], 0.2)
    z_ref[...] = z + fuse
    id_ref[...] = _dot(_lrelu(h * gid[...] + bid[...], 0.1), idw[...])


def _pre0(x, surf_g, p):
    bn = 1000
    pre = 'b0'
    ws = [p['emb'], p['W_fus'], p['b_fus'],
          p[pre + '_in_bng1'], p[pre + '_in_bnb1'], p[pre + '_in_W'], p[pre + '_in_b'],
          p[pre + '_in_bng2'], p[pre + '_in_bnb2'],
          p[pre + '_id_bng'], p[pre + '_id_bnb'], p[pre + '_id_W']]
    dummy = jnp.zeros((N, 1), jnp.float32)
    return pl.pallas_call(
        _pre0_body,
        grid=(N // bn,),
        in_specs=[pl.BlockSpec((bn, 1), lambda i: (i, 0)),
                  pl.BlockSpec((bn, 1), lambda i: (i, 0)),
                  pl.BlockSpec((bn, 128), lambda i: (i, 0))] + [_full_spec(w) for w in ws],
        out_specs=[pl.BlockSpec((bn, W), lambda i: (i, 0)),
                   pl.BlockSpec((bn, CH), lambda i: (i, 0))],
        out_shape=[jax.ShapeDtypeStruct((N, W), jnp.float32),
                   jax.ShapeDtypeStruct((N, CH), jnp.float32)],
    )(x.reshape(N, 1), dummy, surf_g, *ws)


def _pre_body(h_ref, g1, b1, inw, inb, g2, b2, z_ref):
    z = _lrelu(h_ref[...] * g1[...] + b1[...], 0.2)
    z = _dot(z, inw[...]) + inb[...]
    z_ref[...] = _lrelu(z * g2[...] + b2[...], 0.2)


def _pre(h, p, i):
    n = h.shape[0]
    bn = min(n, 1024)
    pre = 'b%d' % i
    ws = [p[pre + '_in_bng1'], p[pre + '_in_bnb1'], p[pre + '_in_W'], p[pre + '_in_b'],
          p[pre + '_in_bng2'], p[pre + '_in_bnb2']]
    return pl.pallas_call(
        _pre_body,
        grid=(pl.cdiv(n, bn),),
        in_specs=[pl.BlockSpec((bn, CH), lambda i: (i, 0))] + [_full_spec(w) for w in ws],
        out_specs=pl.BlockSpec((bn, W), lambda i: (i, 0)),
        out_shape=jax.ShapeDtypeStruct((n, W), jnp.float32),
    )(h, *ws)


def _post_body(p0_ref, p1_ref, deg_ref, id_ref, wc, go, bo, outw, h_ref, *, pool):
    agg = (p0_ref[...] + p1_ref[...]) / jnp.maximum(deg_ref[...], 1.0)
    z = _dot(agg, wc[...])
    z = _lrelu(z * go[...] + bo[...], 0.1)
    hn = _dot(z, outw[...]) + id_ref[...]
    if pool:
        bn = hn.shape[0]
        hp = hn.reshape(bn // 2, 2, CH)
        h_ref[...] = (hp[:, 0, :] + hp[:, 1, :]) * 0.5
    else:
        h_ref[...] = hn


def _post(p0, p1, deg, ident, p, i, pool):
    n = p0.shape[0]
    bn = min(n, 1024)
    pre = 'b%d' % i
    ws = [p[pre + '_Wc'], p[pre + '_out_bng'], p[pre + '_out_bnb'], p[pre + '_out_W']]
    nout = n // 2 if pool else n
    bout = bn // 2 if pool else bn
    return pl.pallas_call(
        functools.partial(_post_body, pool=pool),
        grid=(pl.cdiv(n, bn),),
        in_specs=[pl.BlockSpec((bn, W), lambda i: (i, 0)),
                  pl.BlockSpec((bn, W), lambda i: (i, 0)),
                  pl.BlockSpec((bn, W), lambda i: (i, 0)),
                  pl.BlockSpec((bn, CH), lambda i: (i, 0))] + [_full_spec(w) for w in ws],
        out_specs=pl.BlockSpec((bout, CH), lambda i: (i, 0)),
        out_shape=jax.ShapeDtypeStruct((nout, CH), jnp.float32),
    )(p0, p1, deg, ident, *ws)


def _head_body(h_ref, batch_ref, g1, b1, w1, bb1, g2, b2, w2, bb2, out_ref):
    bid = batch_ref[...]
    mask = (bid == lax.broadcasted_iota(jnp.int32, (1, B), 1)).astype(jnp.float32)
    s = lax.dot_general(mask, h_ref[...], (((0,), (0,)), ((), ())),
                        preferred_element_type=jnp.float32)
    cnt = jnp.sum(mask, axis=0)[:, None]
    h = s / jnp.maximum(cnt, 1.0)
    z = _lrelu(h * g1[...] + b1[...], 0.2)
    z = _dot(z, w1[...]) + bb1[...]
    z = _lrelu(z * g2[...] + b2[...], 0.2)
    out_ref[...] = _dot(z, w2[...]) + bb2[...]


def _head(h, batch, p):
    n = h.shape[0]
    ws = [p['cls_bng1'], p['cls_bnb1'], p['cls_W1'], p['cls_b1'],
          p['cls_bng2'], p['cls_bnb2'], p['cls_W2'], p['cls_b2']]
    return pl.pallas_call(
        _head_body,
        in_specs=[pl.BlockSpec((n, CH), lambda: (0, 0)),
                  pl.BlockSpec((n, 1), lambda: (0, 0))] + [_full_spec(w) for w in ws],
        out_specs=pl.BlockSpec((B, 384), lambda: (0, 0)),
        out_shape=jax.ShapeDtypeStruct((B, 384), jnp.float32),
    )(h, batch.reshape(n, 1), *ws)


def kernel(x, pos, seq, ori, batch, edge_index, features, chem, geo,
           surf2struct, params):
    p = params
    out_surf = _surface(chem, geo, features, p)
    surf_g = _gather_sc(out_surf, surf2struct)

    g = jnp.concatenate([pos, seq, ori], axis=-1)
    src0, dst0 = edge_index[0], edge_index[1]

    h = None
    ident = None
    n = N
    for i in range(8):
        lvl = (i // 2)
        if i % 2 == 0:
            e_src = src0 // (1 << lvl)
            e_dst = dst0 // (1 << lvl)
            g_pad = jnp.pad(g, ((0, 0), (0, 128 - 13)))
            gs16, gd16 = _geo_sc(g_pad, e_src, e_dst)
            kin = _kin_tc(gs16, gd16)
            if i == 0:
                deg = _deg_sc(e_dst, n)

        pre = 'b%d' % i
        if i == 0:
            z, ident = _pre0(x, surf_g, p)
        else:
            z = _pre(h, p, i)
            ident = h

        kern = _edge_kern(kin, p[pre + '_Wk1'], p[pre + '_bk1'], p[pre + '_Wk2'])
        p0, p1 = _conv_sc(z, kern, e_src, e_dst, n)

        pool = (i % 2 == 1) and i != 7
        h = _post(p0, p1, deg, ident, p, i, pool)
        if pool:
            g = (g[0::2] + g[1::2]) * 0.5
            deg = deg[0::2] + deg[1::2]
            batch = batch[0::2]
            n = n // 2

    return _head(h, batch, p)
